# baseline (speedup 1.0000x reference)
import functools

import jax
import jax.numpy as jnp
from jax.experimental import pallas as pl
from jax.experimental.pallas import tpu as pltpu


def _body(x_ref, dt_ref, o_ref, tok_ref, *, C, GG):
    xb = x_ref[0]
    mx = jnp.max(xb, axis=0)
    rev = (C - jax.lax.broadcasted_iota(jnp.int32, xb.shape, 0)).astype(
        jnp.float32
    )
    hit = jnp.where(xb == mx[None], rev, 0.0)
    tok = (C - jnp.max(hit, axis=0)).astype(jnp.int32)
    for r in range(tok.shape[0]):
        tok_ref[0, pl.ds(r * 128, 128)] = tok[r, :]
    tok_row = tok_ref[0, :][None, :]
    iota2 = jax.lax.broadcasted_iota(jnp.int32, (C, GG), 0)
    onehot = jnp.where(iota2 == tok_row, 1.0, 0.0).astype(
        jnp.bfloat16
    )
    o_ref[0] = jax.lax.dot(
        dt_ref[...], onehot, preferred_element_type=jnp.float32
    )


def kernel(x, dictionary):
    B, C, G, G2 = x.shape
    E = dictionary.shape[1]
    GG = G * G2
    xv = x.reshape(B, C, 8, GG // 8)
    dict_t = dictionary.T.astype(jnp.bfloat16)
    out = pl.pallas_call(
        functools.partial(_body, C=C, GG=GG),
        grid=(B,),
        in_specs=[
            pl.BlockSpec((1, C, 8, GG // 8), lambda b: (b, 0, 0, 0)),
            pl.BlockSpec((E, C), lambda b: (0, 0)),
        ],
        out_specs=pl.BlockSpec((1, E, GG), lambda b: (b, 0, 0)),
        out_shape=jax.ShapeDtypeStruct((B, E, GG), jnp.float32),
        scratch_shapes=[pltpu.VMEM((1, GG), jnp.int32)],
    )(xv, dict_t)
    return out.reshape(B, E, G, G2)


# speedup vs baseline: 1.0004x; 1.0004x over previous
import functools

import jax
import jax.numpy as jnp
from jax.experimental import pallas as pl
from jax.experimental.pallas import tpu as pltpu


def _body(x_ref, dt_ref, o_ref, tok_ref, *, C, GG):
    xb = x_ref[0]
    mx = jnp.max(xb, axis=0)
    rev = (C - jax.lax.broadcasted_iota(jnp.int32, xb.shape, 0)).astype(
        jnp.float32
    )
    hit = jnp.where(xb == mx[None], rev, 0.0)
    tok = (C - jnp.max(hit, axis=0)).astype(jnp.int32)
    for r in range(tok.shape[0]):
        tok_ref[0, pl.ds(r * 128, 128)] = tok[r, :]
    tok_row = tok_ref[0, :][None, :]
    iota2 = jax.lax.broadcasted_iota(jnp.int32, (C, GG), 0)
    onehot = jnp.where(iota2 == tok_row, 1.0, 0.0).astype(
        jnp.bfloat16
    )
    o_ref[0] = jax.lax.dot(
        dt_ref[...], onehot, preferred_element_type=jnp.float32
    )


def kernel(x, dictionary):
    B, C, G, G2 = x.shape
    E = dictionary.shape[1]
    GG = G * G2
    xv = x.reshape(B, C, 8, GG // 8)
    dict_t = dictionary.T.astype(jnp.bfloat16)
    out = pl.pallas_call(
        functools.partial(_body, C=C, GG=GG),
        grid=(B,),
        in_specs=[
            pl.BlockSpec((1, C, 8, GG // 8), lambda b: (b, 0, 0, 0)),
            pl.BlockSpec((E, C), lambda b: (0, 0)),
        ],
        out_specs=pl.BlockSpec((1, E, GG), lambda b: (b, 0, 0)),
        out_shape=jax.ShapeDtypeStruct((B, E, GG), jnp.float32),
        scratch_shapes=[pltpu.VMEM((1, GG), jnp.int32)],
        compiler_params=pltpu.CompilerParams(
            dimension_semantics=("parallel",)
        ),
    )(xv, dict_t)
    return out.reshape(B, E, G, G2)


# speedup vs baseline: 1.1488x; 1.1484x over previous
import functools

import jax
import jax.numpy as jnp
from jax.experimental import pallas as pl
from jax.experimental.pallas import tpu as pltpu

_BB = 2


def _body(x_ref, dt_ref, o_ref, tok_ref, *, C, GG):
    xb = x_ref[...]
    mx = jnp.max(xb, axis=1)
    rev = (C - jax.lax.broadcasted_iota(jnp.int32, xb.shape, 1)).astype(
        jnp.float32
    )
    hit = jnp.where(xb == mx[:, None], rev, 0.0)
    tok = (C - jnp.max(hit, axis=1)).astype(jnp.int32)
    for b in range(_BB):
        for r in range(tok.shape[1]):
            tok_ref[0, pl.ds(b * GG + r * 128, 128)] = tok[b, r, :]
    tok_row = tok_ref[0, :][None, :]
    iota2 = jax.lax.broadcasted_iota(jnp.int32, (C, _BB * GG), 0)
    onehot = jnp.where(iota2 == tok_row, 1.0, 0.0).astype(
        jnp.bfloat16
    )
    mm = jax.lax.dot(
        dt_ref[...], onehot, preferred_element_type=jnp.float32
    )
    for b in range(_BB):
        o_ref[b] = mm[:, b * GG : (b + 1) * GG]


def kernel(x, dictionary):
    B, C, G, G2 = x.shape
    E = dictionary.shape[1]
    GG = G * G2
    xv = x.reshape(B, C, 8, GG // 8)
    dict_t = dictionary.T.astype(jnp.bfloat16)
    out = pl.pallas_call(
        functools.partial(_body, C=C, GG=GG),
        grid=(B // _BB,),
        in_specs=[
            pl.BlockSpec((_BB, C, 8, GG // 8), lambda b: (b, 0, 0, 0)),
            pl.BlockSpec((E, C), lambda b: (0, 0)),
        ],
        out_specs=pl.BlockSpec((_BB, E, GG), lambda b: (b, 0, 0)),
        out_shape=jax.ShapeDtypeStruct((B, E, GG), jnp.float32),
        scratch_shapes=[pltpu.VMEM((1, _BB * GG), jnp.int32)],
        compiler_params=pltpu.CompilerParams(
            dimension_semantics=("parallel",)
        ),
    )(xv, dict_t)
    return out.reshape(B, E, G, G2)


# speedup vs baseline: 1.2014x; 1.0458x over previous
import functools

import jax
import jax.numpy as jnp
from jax.experimental import pallas as pl
from jax.experimental.pallas import tpu as pltpu

_BB = 4


def _body(x_ref, dt_ref, o_ref, tok_ref, *, C, GG):
    xb = x_ref[...]
    mx = jnp.max(xb, axis=1)
    rev = (C - jax.lax.broadcasted_iota(jnp.int32, xb.shape, 1)).astype(
        jnp.float32
    )
    hit = jnp.where(xb == mx[:, None], rev, 0.0)
    tok = (C - jnp.max(hit, axis=1)).astype(jnp.int32)
    for b in range(_BB):
        for r in range(tok.shape[1]):
            tok_ref[0, pl.ds(b * GG + r * 128, 128)] = tok[b, r, :]
    tok_row = tok_ref[0, :][None, :]
    iota2 = jax.lax.broadcasted_iota(jnp.int32, (C, _BB * GG), 0)
    onehot = jnp.where(iota2 == tok_row, 1.0, 0.0).astype(
        jnp.bfloat16
    )
    mm = jax.lax.dot(
        dt_ref[...], onehot, preferred_element_type=jnp.float32
    )
    for b in range(_BB):
        o_ref[b] = mm[:, b * GG : (b + 1) * GG]


def kernel(x, dictionary):
    B, C, G, G2 = x.shape
    E = dictionary.shape[1]
    GG = G * G2
    xv = x.reshape(B, C, 8, GG // 8)
    dict_t = dictionary.T.astype(jnp.bfloat16)
    out = pl.pallas_call(
        functools.partial(_body, C=C, GG=GG),
        grid=(B // _BB,),
        in_specs=[
            pl.BlockSpec((_BB, C, 8, GG // 8), lambda b: (b, 0, 0, 0)),
            pl.BlockSpec((E, C), lambda b: (0, 0)),
        ],
        out_specs=pl.BlockSpec((_BB, E, GG), lambda b: (b, 0, 0)),
        out_shape=jax.ShapeDtypeStruct((B, E, GG), jnp.float32),
        scratch_shapes=[pltpu.VMEM((1, _BB * GG), jnp.int32)],
        compiler_params=pltpu.CompilerParams(
            dimension_semantics=("parallel",)
        ),
    )(xv, dict_t)
    return out.reshape(B, E, G, G2)
